# baseline (speedup 1.0000x reference)
import jax
import jax.numpy as jnp
from jax.experimental import pallas as pl
from jax.experimental.pallas import tpu as pltpu

_IN_CH = 96
_CP = 128
_HID = 192
_OUT = 85
_GN_EPS = 1e-05
_STRIDES = (8, 16, 32, 64, 128)
_SIZES = ((64, 64), (32, 32), (16, 16), (8, 8), (4, 4))


def _one_level(x, w3_ref, rows_ref, m_ref, wf_ref, fb_ref,
               cls_ref, cent_ref, reg_ref, H, W, stride):
    hw = H * W
    x2 = jnp.transpose(x)
    wmod = jax.lax.broadcasted_iota(jnp.int32, (hw, _IN_CH), 0) % W
    xm0 = jnp.where(wmod == W - 1, jnp.bfloat16(0), x2)
    xm2 = jnp.where(wmod == 0, jnp.bfloat16(0), x2)
    zb = jnp.zeros((W, _IN_CH), jnp.bfloat16)
    z1 = jnp.zeros((1, _IN_CH), jnp.bfloat16)
    zlane = jnp.zeros((hw + 2 * W, _CP - _IN_CH), jnp.bfloat16)
    slot0 = jnp.concatenate([zb, z1, xm0[0:hw - 1], zb], axis=0)
    slot1 = jnp.concatenate([zb, x2, zb], axis=0)
    slot2 = jnp.concatenate([zb, xm2[1:hw], zb, z1], axis=0)
    xcat = jnp.concatenate(
        [slot0, zlane, slot1, zlane, slot2, zlane], axis=1)
    acc = jnp.zeros((hw, _HID), dtype=jnp.float32)
    for ky in range(3):
        xs = xcat[ky * W:ky * W + hw]
        acc = acc + jnp.dot(xs, w3_ref[ky],
                            preferred_element_type=jnp.float32)
    bias = rows_ref[0:1]
    gamma = rows_ref[1:2]
    beta = rows_ref[2:3]
    s1 = jnp.sum(acc, axis=0, keepdims=True)
    s2 = jnp.sum(acc * acc, axis=0, keepdims=True)
    t1 = s1 + hw * bias
    t2 = s2 + (2.0 * bias) * s1 + hw * (bias * bias)
    g1 = jnp.dot(t1, m_ref[...], preferred_element_type=jnp.float32)
    g2 = jnp.dot(t2, m_ref[...], preferred_element_type=jnp.float32)
    n = 3.0 * hw
    mean = g1 / n
    var = g2 / n - mean * mean
    scale = jax.lax.rsqrt(var + _GN_EPS) * gamma
    shift = (bias - mean) * scale + beta
    h = acc * scale + shift
    h = (h * jax.nn.sigmoid(h)).astype(jnp.bfloat16)
    yt = jax.lax.dot_general(wf_ref[...], h, (((0,), (1,)), ((), ())),
                             preferred_element_type=jnp.float32)
    yt = yt + fb_ref[...]
    cls_ref[0] = yt[0:80]
    cent_ref[0] = yt[80:81]
    reg_ref[0] = jnp.maximum(yt[81:85] * float(stride), 0.0)


def _fused_kernel(x0, x1, x2, x3, x4, w3_ref, rows_ref, m_ref, wf_ref,
                  fb_ref, *out_refs):
    xs = (x0, x1, x2, x3, x4)
    for i, ((H, W), stride, xr) in enumerate(zip(_SIZES, _STRIDES, xs)):
        _one_level(xr[0], w3_ref, rows_ref, m_ref, wf_ref, fb_ref,
                   out_refs[3 * i], out_refs[3 * i + 1], out_refs[3 * i + 2],
                   H, W, stride)


def kernel(fpn0, fpn1, fpn2, fpn3, fpn4,
           cls_w, cls_b, cls_g, cls_beta, cls_fw, cls_fb,
           reg_w, reg_b, reg_g, reg_beta, reg_fw, reg_fb):
    fpn = (fpn0, fpn1, fpn2, fpn3, fpn4)
    B = fpn0.shape[0]

    def taps(w):
        return jnp.transpose(w, (2, 3, 1, 0))
    w3 = jnp.concatenate([taps(cls_w), taps(reg_w)], axis=-1)
    w3 = jnp.pad(w3, ((0, 0), (0, 0), (0, _CP - _IN_CH), (0, 0)))
    w3 = w3.reshape(3, 3 * _CP, _HID).astype(jnp.bfloat16)
    rows = jnp.stack([
        jnp.concatenate([cls_b, reg_b]),
        jnp.concatenate([cls_g, reg_g]),
        jnp.concatenate([cls_beta, reg_beta]),
    ], axis=0)
    ids = jnp.arange(_HID) // 3
    m = (ids[:, None] == ids[None, :]).astype(jnp.float32)
    wf = jnp.zeros((_HID, _OUT), jnp.float32)
    wf = wf.at[:_IN_CH, :80].set(jnp.transpose(cls_fw.reshape(80, _IN_CH)))
    wf = wf.at[_IN_CH:, 80:].set(jnp.transpose(reg_fw.reshape(5, _IN_CH)))
    wf = wf.astype(jnp.bfloat16)
    fb = jnp.concatenate([cls_fb, reg_fb])[:, None]

    xps, in_specs, out_specs, out_shapes = [], [], [], []
    for (H, W), x in zip(_SIZES, fpn):
        hw = H * W
        xps.append(x.reshape(B, _IN_CH, hw).astype(jnp.bfloat16))
        in_specs.append(
            pl.BlockSpec((1, _IN_CH, hw), lambda b: (b, 0, 0)))
        for c in (80, 1, 4):
            out_specs.append(pl.BlockSpec((1, c, hw), lambda b: (b, 0, 0)))
            out_shapes.append(jax.ShapeDtypeStruct((B, c, hw), jnp.float32))
    in_specs += [
        pl.BlockSpec((3, 3 * _CP, _HID), lambda b: (0, 0, 0)),
        pl.BlockSpec((3, _HID), lambda b: (0, 0)),
        pl.BlockSpec((_HID, _HID), lambda b: (0, 0)),
        pl.BlockSpec((_HID, _OUT), lambda b: (0, 0)),
        pl.BlockSpec((_OUT, 1), lambda b: (0, 0)),
    ]

    outs = pl.pallas_call(
        _fused_kernel,
        grid=(B,),
        in_specs=in_specs,
        out_specs=out_specs,
        out_shape=out_shapes,
        compiler_params=pltpu.CompilerParams(
            dimension_semantics=("parallel",)),
    )(*xps, w3, rows, m, wf, fb)

    cls_out, reg_out, cent_out = [], [], []
    for i, (H, W) in enumerate(_SIZES):
        cls_out.append(outs[3 * i].reshape(B, 80, H, W))
        cent_out.append(outs[3 * i + 1].reshape(B, 1, H, W))
        reg_out.append(outs[3 * i + 2].reshape(B, 4, H, W))
    return tuple(cls_out) + tuple(reg_out) + tuple(cent_out)


# speedup vs baseline: 1.0372x; 1.0372x over previous
import jax
import jax.numpy as jnp
from jax.experimental import pallas as pl
from jax.experimental.pallas import tpu as pltpu

_IN_CH = 96
_CP = 128
_HID = 192
_OUT = 85
_GN_EPS = 1e-05
_STRIDES = (8, 16, 32, 64, 128)
_SIZES = ((64, 64), (32, 32), (16, 16), (8, 8), (4, 4))
_SEG_OFF = (0, 1024, 1280, 1408)
_SEG_LEN = 1536


def _head_pipeline(x2, w3_ref, rows_ref, m_ref, wf_ref, fb_ref, H, W, stride):
    hw = H * W
    wmod = jax.lax.broadcasted_iota(jnp.int32, (hw, _IN_CH), 0) % W
    xm0 = jnp.where(wmod == W - 1, jnp.bfloat16(0), x2)
    xm2 = jnp.where(wmod == 0, jnp.bfloat16(0), x2)
    zb = jnp.zeros((W, _IN_CH), jnp.bfloat16)
    z1 = jnp.zeros((1, _IN_CH), jnp.bfloat16)
    zlane = jnp.zeros((hw + 2 * W, _CP - _IN_CH), jnp.bfloat16)
    slot0 = jnp.concatenate([zb, z1, xm0[0:hw - 1], zb], axis=0)
    slot1 = jnp.concatenate([zb, x2, zb], axis=0)
    slot2 = jnp.concatenate([zb, xm2[1:hw], zb, z1], axis=0)
    xcat = jnp.concatenate(
        [slot0, zlane, slot1, zlane, slot2, zlane], axis=1)
    acc = jnp.zeros((hw, _HID), dtype=jnp.float32)
    for ky in range(3):
        xs = xcat[ky * W:ky * W + hw]
        acc = acc + jnp.dot(xs, w3_ref[ky],
                            preferred_element_type=jnp.float32)
    bias = rows_ref[0:1]
    gamma = rows_ref[1:2]
    beta = rows_ref[2:3]
    s1 = jnp.sum(acc, axis=0, keepdims=True)
    s2 = jnp.sum(acc * acc, axis=0, keepdims=True)
    t1 = s1 + hw * bias
    t2 = s2 + (2.0 * bias) * s1 + hw * (bias * bias)
    g1 = jnp.dot(t1, m_ref[...], preferred_element_type=jnp.float32)
    g2 = jnp.dot(t2, m_ref[...], preferred_element_type=jnp.float32)
    n = 3.0 * hw
    mean = g1 / n
    var = g2 / n - mean * mean
    scale = jax.lax.rsqrt(var + _GN_EPS) * gamma
    shift = (bias - mean) * scale + beta
    h = acc * scale + shift
    h = (h * jax.nn.sigmoid(h)).astype(jnp.bfloat16)
    yt = jax.lax.dot_general(wf_ref[...], h, (((0,), (1,)), ((), ())),
                             preferred_element_type=jnp.float32)
    yt = yt + fb_ref[...]
    return yt


def _fused_kernel(x0, xsml, w3_ref, rows_ref, m_ref, wf_ref, fb_ref,
                  cls0, cent0, reg0, clss, cents, regs):
    H, W = _SIZES[0]
    yt = _head_pipeline(jnp.transpose(x0[0]), w3_ref, rows_ref, m_ref,
                        wf_ref, fb_ref, H, W, _STRIDES[0])
    cls0[0] = yt[0:80]
    cent0[0] = yt[80:81]
    reg0[0] = jnp.maximum(yt[81:85] * float(_STRIDES[0]), 0.0)
    for i, (H, W) in enumerate(_SIZES[1:]):
        hw = H * W
        off = _SEG_OFF[i]
        x2 = jnp.transpose(xsml[0][:, off:off + hw])
        yt = _head_pipeline(x2, w3_ref, rows_ref, m_ref, wf_ref, fb_ref,
                            H, W, _STRIDES[i + 1])
        clss[0, :, off:off + hw] = yt[0:80]
        cents[0, :, off:off + hw] = yt[80:81]
        regs[0, :, off:off + hw] = jnp.maximum(
            yt[81:85] * float(_STRIDES[i + 1]), 0.0)


def kernel(fpn0, fpn1, fpn2, fpn3, fpn4,
           cls_w, cls_b, cls_g, cls_beta, cls_fw, cls_fb,
           reg_w, reg_b, reg_g, reg_beta, reg_fw, reg_fb):
    B = fpn0.shape[0]

    def taps(w):
        return jnp.transpose(w, (2, 3, 1, 0))
    w3 = jnp.concatenate([taps(cls_w), taps(reg_w)], axis=-1)
    w3 = jnp.pad(w3, ((0, 0), (0, 0), (0, _CP - _IN_CH), (0, 0)))
    w3 = w3.reshape(3, 3 * _CP, _HID).astype(jnp.bfloat16)
    rows = jnp.stack([
        jnp.concatenate([cls_b, reg_b]),
        jnp.concatenate([cls_g, reg_g]),
        jnp.concatenate([cls_beta, reg_beta]),
    ], axis=0)
    ids = jnp.arange(_HID) // 3
    m = (ids[:, None] == ids[None, :]).astype(jnp.float32)
    wf = jnp.zeros((_HID, _OUT), jnp.float32)
    wf = wf.at[:_IN_CH, :80].set(jnp.transpose(cls_fw.reshape(80, _IN_CH)))
    wf = wf.at[_IN_CH:, 80:].set(jnp.transpose(reg_fw.reshape(5, _IN_CH)))
    wf = wf.astype(jnp.bfloat16)
    fb = jnp.concatenate([cls_fb, reg_fb])[:, None]

    hw0 = _SIZES[0][0] * _SIZES[0][1]
    x0 = fpn0.reshape(B, _IN_CH, hw0).astype(jnp.bfloat16)
    sml = []
    for i, (x, (H, W)) in enumerate(zip((fpn1, fpn2, fpn3, fpn4),
                                        _SIZES[1:])):
        hw = H * W
        seg_end = _SEG_OFF[i + 1] if i < 3 else _SEG_LEN
        seg = seg_end - _SEG_OFF[i]
        xf = x.reshape(B, _IN_CH, hw)
        if seg > hw:
            xf = jnp.pad(xf, ((0, 0), (0, 0), (0, seg - hw)))
        sml.append(xf)
    xsml = jnp.concatenate(sml, axis=2).astype(jnp.bfloat16)

    in_specs = [
        pl.BlockSpec((1, _IN_CH, hw0), lambda b: (b, 0, 0)),
        pl.BlockSpec((1, _IN_CH, _SEG_LEN), lambda b: (b, 0, 0)),
        pl.BlockSpec((3, 3 * _CP, _HID), lambda b: (0, 0, 0)),
        pl.BlockSpec((3, _HID), lambda b: (0, 0)),
        pl.BlockSpec((_HID, _HID), lambda b: (0, 0)),
        pl.BlockSpec((_HID, _OUT), lambda b: (0, 0)),
        pl.BlockSpec((_OUT, 1), lambda b: (0, 0)),
    ]
    out_specs, out_shapes = [], []
    for hw in (hw0, _SEG_LEN):
        for c in (80, 1, 4):
            out_specs.append(pl.BlockSpec((1, c, hw), lambda b: (b, 0, 0)))
            out_shapes.append(jax.ShapeDtypeStruct((B, c, hw), jnp.float32))

    outs = pl.pallas_call(
        _fused_kernel,
        grid=(B,),
        in_specs=in_specs,
        out_specs=out_specs,
        out_shape=out_shapes,
        compiler_params=pltpu.CompilerParams(
            dimension_semantics=("parallel",)),
    )(x0, xsml, w3, rows, m, wf, fb)

    cls_out = [outs[0].reshape(B, 80, *_SIZES[0])]
    cent_out = [outs[1].reshape(B, 1, *_SIZES[0])]
    reg_out = [outs[2].reshape(B, 4, *_SIZES[0])]
    for i, (H, W) in enumerate(_SIZES[1:]):
        hw = H * W
        off = _SEG_OFF[i]
        cls_out.append(outs[3][:, :, off:off + hw].reshape(B, 80, H, W))
        cent_out.append(outs[4][:, :, off:off + hw].reshape(B, 1, H, W))
        reg_out.append(outs[5][:, :, off:off + hw].reshape(B, 4, H, W))
    return tuple(cls_out) + tuple(reg_out) + tuple(cent_out)


# speedup vs baseline: 1.0739x; 1.0354x over previous
import numpy as np

import jax
import jax.numpy as jnp
from jax.experimental import pallas as pl
from jax.experimental.pallas import tpu as pltpu

_IN_CH = 96
_CP = 128
_HID = 192
_OUT = 85
_GN_EPS = 1e-05
_STRIDES = (8, 16, 32, 64, 128)
_SIZES = ((64, 64), (32, 32), (16, 16), (8, 8), (4, 4))
_SEG_OFF = (0, 1024, 1280, 1408)
_SEG_LEN = 1536
_OUTLEN = (1024, 256, 128, 16)
_M_SMALL = 1424
_SOFF = (0, 1088, 1376, 1456)
_SLOT_ROWS = 1520


def _conv_gn_head(xcat, seg_bounds, w3_ref, rows_ref, m_ref, wf_ref, fb_ref,
                  ky_slices):
    acc = None
    for ky in range(3):
        pieces = [xcat[s:s + l] for s, l in ky_slices[ky]]
        xs = pieces[0] if len(pieces) == 1 else jnp.concatenate(pieces, 0)
        d = jnp.dot(xs, w3_ref[ky], preferred_element_type=jnp.float32)
        acc = d if acc is None else acc + d
    bias = rows_ref[0:1]
    gamma = rows_ref[1:2]
    beta = rows_ref[2:3]
    acc2 = acc * acc
    t_rows = []
    for r0, nr in seg_bounds:
        s1 = jnp.sum(acc[r0:r0 + nr], axis=0, keepdims=True)
        s2 = jnp.sum(acc2[r0:r0 + nr], axis=0, keepdims=True)
        t_rows.append(s1 + nr * bias)
        t_rows.append(s2 + (2.0 * bias) * s1 + nr * (bias * bias))
    t = jnp.concatenate(t_rows, axis=0) if len(t_rows) > 2 else None
    if t is None:
        t1, t2 = t_rows
        g1 = jnp.dot(t1, m_ref[...], preferred_element_type=jnp.float32)
        g2 = jnp.dot(t2, m_ref[...], preferred_element_type=jnp.float32)
        gs = [(g1, g2)]
    else:
        g = jnp.dot(t, m_ref[...], preferred_element_type=jnp.float32)
        gs = [(g[2 * i:2 * i + 1], g[2 * i + 1:2 * i + 2])
              for i in range(len(seg_bounds))]
    scs, shs = [], []
    for (r0, nr), (g1, g2) in zip(seg_bounds, gs):
        n = 3.0 * nr
        mean = g1 / n
        var = g2 / n - mean * mean
        sc = jax.lax.rsqrt(var + _GN_EPS) * gamma
        scs.append(sc)
        shs.append((bias - mean) * sc + beta)
    if len(seg_bounds) == 1:
        h = acc * scs[0] + shs[0]
    else:
        spans = []
        bounds = [r0 for r0, _ in seg_bounds] + [acc.shape[0]]
        for i in range(len(seg_bounds)):
            spans.append(bounds[i + 1] - bounds[i])
        scale_f = jnp.concatenate(
            [jnp.broadcast_to(s, (sp, _HID)) for s, sp in zip(scs, spans)], 0)
        shift_f = jnp.concatenate(
            [jnp.broadcast_to(s, (sp, _HID)) for s, sp in zip(shs, spans)], 0)
        h = acc * scale_f + shift_f
    h = (h * jax.nn.sigmoid(h)).astype(jnp.bfloat16)
    yt = jax.lax.dot_general(wf_ref[...], h, (((0,), (1,)), ((), ())),
                             preferred_element_type=jnp.float32)
    return yt + fb_ref[...]


def _slots(x2, xm0, xm2, regions):
    p0, p1, p2 = [], [], []
    for src, hw, W in regions:
        zb = jnp.zeros((W, _IN_CH), jnp.bfloat16)
        z1 = jnp.zeros((1, _IN_CH), jnp.bfloat16)
        p0 += [zb, z1, xm0[src:src + hw - 1], zb]
        p1 += [zb, x2[src:src + hw], zb]
        p2 += [zb, xm2[src + 1:src + hw], zb, z1]
    total = sum(2 * W + hw for _, hw, W in regions)
    rows = max(_SLOT_ROWS, total) if len(regions) > 1 else total
    if rows > total:
        ztail = jnp.zeros((rows - total, _IN_CH), jnp.bfloat16)
        p0.append(ztail)
        p1.append(ztail)
        p2.append(ztail)
    zlane = jnp.zeros((rows, _CP - _IN_CH), jnp.bfloat16)
    return jnp.concatenate(
        [jnp.concatenate(p0, 0), zlane, jnp.concatenate(p1, 0), zlane,
         jnp.concatenate(p2, 0), zlane], axis=1)


def _fused_kernel(x0, xsml, m0_ref, m2_ref, sv_ref, w3_ref, rows_ref, m_ref,
                  wf_ref, fb_ref, cls0, cent0, reg0, clss, cents, regs):
    H, W = _SIZES[0]
    hw = H * W
    x2 = jnp.transpose(x0[0])
    wmod = jax.lax.broadcasted_iota(jnp.int32, (hw, _IN_CH), 0) % W
    xm0 = jnp.where(wmod == W - 1, jnp.bfloat16(0), x2)
    xm2 = jnp.where(wmod == 0, jnp.bfloat16(0), x2)
    xcat = _slots(x2, xm0, xm2, [(0, hw, W)])
    ky_slices = [[(ky * W, hw)] for ky in range(3)]
    yt = _conv_gn_head(xcat, [(0, hw)], w3_ref, rows_ref, m_ref, wf_ref,
                       fb_ref, ky_slices)
    cls0[0] = yt[0:80]
    cent0[0] = yt[80:81]
    reg0[0] = jnp.maximum(yt[81:85] * float(_STRIDES[0]), 0.0)
    x2s = jnp.transpose(xsml[0])
    xm0s = x2s * m0_ref[...]
    xm2s = x2s * m2_ref[...]
    regions = [(_SEG_OFF[i], Hs * Ws, Ws)
               for i, (Hs, Ws) in enumerate(_SIZES[1:])]
    xcat = _slots(x2s, xm0s, xm2s, regions)
    ky_slices = [[(_SOFF[i] + ky * Ws, _OUTLEN[i])
                  for i, (_, Ws) in enumerate(_SIZES[1:])]
                 for ky in range(3)]
    seg_bounds = [(_SEG_OFF[i], Hs * Ws)
                  for i, (Hs, Ws) in enumerate(_SIZES[1:])]
    yt = _conv_gn_head(xcat, seg_bounds, w3_ref, rows_ref, m_ref, wf_ref,
                       fb_ref, ky_slices)
    clss[0, :, 0:_M_SMALL] = yt[0:80]
    cents[0, :, 0:_M_SMALL] = yt[80:81]
    regs[0, :, 0:_M_SMALL] = jnp.maximum(yt[81:85] * sv_ref[...], 0.0)


def kernel(fpn0, fpn1, fpn2, fpn3, fpn4,
           cls_w, cls_b, cls_g, cls_beta, cls_fw, cls_fb,
           reg_w, reg_b, reg_g, reg_beta, reg_fw, reg_fb):
    B = fpn0.shape[0]

    def taps(w):
        return jnp.transpose(w, (2, 3, 1, 0))
    w3 = jnp.concatenate([taps(cls_w), taps(reg_w)], axis=-1)
    w3 = jnp.pad(w3, ((0, 0), (0, 0), (0, _CP - _IN_CH), (0, 0)))
    w3 = w3.reshape(3, 3 * _CP, _HID).astype(jnp.bfloat16)
    rows = jnp.stack([
        jnp.concatenate([cls_b, reg_b]),
        jnp.concatenate([cls_g, reg_g]),
        jnp.concatenate([cls_beta, reg_beta]),
    ], axis=0)
    ids = jnp.arange(_HID) // 3
    m = (ids[:, None] == ids[None, :]).astype(jnp.float32)
    wf = jnp.zeros((_HID, _OUT), jnp.float32)
    wf = wf.at[:_IN_CH, :80].set(jnp.transpose(cls_fw.reshape(80, _IN_CH)))
    wf = wf.at[_IN_CH:, 80:].set(jnp.transpose(reg_fw.reshape(5, _IN_CH)))
    wf = wf.astype(jnp.bfloat16)
    fb = jnp.concatenate([cls_fb, reg_fb])[:, None]

    m0_np = np.zeros((_SEG_LEN, 1), np.float32)
    m2_np = np.zeros((_SEG_LEN, 1), np.float32)
    sv_np = np.ones((1, _SEG_LEN), np.float32)
    for i, (H, W) in enumerate(_SIZES[1:]):
        off, hw = _SEG_OFF[i], H * W
        q = np.arange(hw)
        m0_np[off + q, 0] = (q % W != W - 1)
        m2_np[off + q, 0] = (q % W != 0)
    for i in range(4):
        end = _SEG_OFF[i + 1] if i < 3 else _SEG_LEN
        sv_np[0, _SEG_OFF[i]:end] = float(_STRIDES[i + 1])
    m0c = jnp.asarray(m0_np, jnp.bfloat16)
    m2c = jnp.asarray(m2_np, jnp.bfloat16)
    svc = jnp.asarray(sv_np[:, :_M_SMALL])

    hw0 = _SIZES[0][0] * _SIZES[0][1]
    x0 = fpn0.reshape(B, _IN_CH, hw0).astype(jnp.bfloat16)
    sml = []
    for i, (x, (H, W)) in enumerate(zip((fpn1, fpn2, fpn3, fpn4),
                                        _SIZES[1:])):
        hw = H * W
        seg_end = _SEG_OFF[i + 1] if i < 3 else _SEG_LEN
        seg = seg_end - _SEG_OFF[i]
        xf = x.reshape(B, _IN_CH, hw)
        if seg > hw:
            xf = jnp.pad(xf, ((0, 0), (0, 0), (0, seg - hw)))
        sml.append(xf)
    xsml = jnp.concatenate(sml, axis=2).astype(jnp.bfloat16)

    in_specs = [
        pl.BlockSpec((1, _IN_CH, hw0), lambda b: (b, 0, 0)),
        pl.BlockSpec((1, _IN_CH, _SEG_LEN), lambda b: (b, 0, 0)),
        pl.BlockSpec((_SEG_LEN, 1), lambda b: (0, 0)),
        pl.BlockSpec((_SEG_LEN, 1), lambda b: (0, 0)),
        pl.BlockSpec((1, _M_SMALL), lambda b: (0, 0)),
        pl.BlockSpec((3, 3 * _CP, _HID), lambda b: (0, 0, 0)),
        pl.BlockSpec((3, _HID), lambda b: (0, 0)),
        pl.BlockSpec((_HID, _HID), lambda b: (0, 0)),
        pl.BlockSpec((_HID, _OUT), lambda b: (0, 0)),
        pl.BlockSpec((_OUT, 1), lambda b: (0, 0)),
    ]
    out_specs, out_shapes = [], []
    for hw in (hw0, _SEG_LEN):
        for c in (80, 1, 4):
            out_specs.append(pl.BlockSpec((1, c, hw), lambda b: (b, 0, 0)))
            out_shapes.append(jax.ShapeDtypeStruct((B, c, hw), jnp.float32))

    outs = pl.pallas_call(
        _fused_kernel,
        grid=(B,),
        in_specs=in_specs,
        out_specs=out_specs,
        out_shape=out_shapes,
        compiler_params=pltpu.CompilerParams(
            dimension_semantics=("parallel",)),
    )(x0, xsml, m0c, m2c, svc, w3, rows, m, wf, fb)

    cls_out = [outs[0].reshape(B, 80, *_SIZES[0])]
    cent_out = [outs[1].reshape(B, 1, *_SIZES[0])]
    reg_out = [outs[2].reshape(B, 4, *_SIZES[0])]
    for i, (H, W) in enumerate(_SIZES[1:]):
        hw = H * W
        off = _SEG_OFF[i]
        cls_out.append(outs[3][:, :, off:off + hw].reshape(B, 80, H, W))
        cent_out.append(outs[4][:, :, off:off + hw].reshape(B, 1, H, W))
        reg_out.append(outs[5][:, :, off:off + hw].reshape(B, 4, H, W))
    return tuple(cls_out) + tuple(reg_out) + tuple(cent_out)
